# baseline (speedup 1.0000x reference)
import functools
import jax
import jax.numpy as jnp
from jax import lax
from jax.experimental import pallas as pl
from jax.experimental.pallas import tpu as pltpu
from jax.experimental.pallas import tpu_sc as plsc

H = 4
F = 32
NEG_SLOPE = 0.2

NC, NS, L = 2, 16, 16
NW = NC * NS

N = 10000
E = 320000
NPAD = 10240
EPW = E // NW
BLK = 2000
NBLK = EPW // BLK
GRP = BLK // L
SROW = 125
NSC = BLK // SROW
ACC_PT = NPAD // NS


def _node_proj_body(w_ref, c_ref, x_ref, o_ref):
    o_ref[...] = lax.dot_general(
        w_ref[...], x_ref[...], (((0,), (1,)), ((), ())),
        preferred_element_type=jnp.float32,
    ) + c_ref[...][:, :1]


def _edge_proj_body(w_ref, c_ref, x_ref, o_ref):
    o_ref[...] = lax.dot_general(
        w_ref[...], x_ref[...], (((0,), (0,)), ((), ())),
        preferred_element_type=jnp.float32,
    ) + c_ref[...][:, :1]


def _fin_body(a_ref, b_ref, o_ref):
    a = a_ref[...]
    b = b_ref[...]
    den = a[:, 0:4] + b[:, 0:4]
    ms = a[:, 4:8] + b[:, 4:8]
    o_ref[...] = jnp.where(den > 0, ms / den, 0.0)


def _sc_edge_kernel(tab_hbm, ei_hbm, ee_hbm, zero_hbm, out_hbm,
                    tab_v, sda_v, sdb_v, eea_v, eeb_v, dst2_v, buf_v, acc,
                    sem_a, sem_in):
    cid = lax.axis_index("c")
    sid = lax.axis_index("s")
    wid = cid * NS + sid

    pltpu.sync_copy(tab_hbm.at[:, pl.ds(0, N)], tab_v)

    pltpu.sync_copy(zero_hbm, acc.at[pl.ds(sid * ACC_PT, ACC_PT)])
    plsc.subcore_barrier()

    iota = lax.iota(jnp.int32, L)
    base = wid * EPW
    buf = buf_v
    dst2 = dst2_v
    sds = [sda_v, sdb_v]
    ees = [eea_v, eeb_v]

    def start_inputs(k):
        off = base + k * BLK
        return [
            pltpu.async_copy(ei_hbm.at[:, pl.ds(off, BLK)], sds[k % 2], sem_in),
            pltpu.async_copy(ee_hbm.at[:, pl.ds(off, BLK)], ees[k % 2], sem_in),
        ]

    descs = [None] * NBLK
    in_descs = [None] * NBLK
    in_descs[0] = start_inputs(0)
    for k in range(NBLK):
        for d in in_descs[k]:
            d.wait()
        if k + 1 < NBLK:
            in_descs[k + 1] = start_inputs(k + 1)
        if k >= 1:
            for d in descs[k - 1]:
                d.wait()
        sd_v = sds[k % 2]
        ee_v = ees[k % 2]

        @plsc.parallel_loop(0, GRP, unroll=8)
        def gbody(g):
            s16 = g * L
            src16 = sd_v[0, pl.ds(s16, L)]
            dst16 = sd_v[1, pl.ds(s16, L)]
            pos = s16 + iota
            plsc.store_scatter(dst2, [pos // SROW, pos % SROW], dst16)
            for h in range(H):
                hrow = jnp.full((L,), h, jnp.int32)
                el = plsc.load_gather(tab_v, [hrow, src16])
                er = plsc.load_gather(tab_v, [hrow + H, dst16])
                ee = ee_v[h, pl.ds(s16, L)]
                s = el + ee
                t = s + er
                t = jnp.where(t >= 0.0, t, NEG_SLOPE * t)
                num = jnp.exp(t)
                ns = num * s
                plsc.store_scatter(buf, [pos, hrow], num)
                plsc.store_scatter(buf, [pos, hrow + H], ns)

        descs[k] = [
            pltpu.async_copy(buf.at[pl.ds(j * SROW, SROW)],
                             acc.at[dst2.at[j]], sem_a, add=True)
            for j in range(NSC)
        ]

    for d in descs[NBLK - 1]:
        d.wait()

    plsc.subcore_barrier()
    r0 = sid * ACC_PT
    pltpu.sync_copy(acc.at[pl.ds(r0, ACC_PT)],
                    out_hbm.at[pl.ds(cid * NPAD + r0, ACC_PT)])


_sc_edge_call = functools.partial(
    pl.kernel,
    out_type=jax.ShapeDtypeStruct((NC * NPAD, 8), jnp.float32),
    mesh=plsc.VectorSubcoreMesh(core_axis_name="c", subcore_axis_name="s"),
    compiler_params=pltpu.CompilerParams(needs_layout_passes=False,
                                         use_tc_tiling_on_sc=False),
    scratch_types=[
        pltpu.VMEM((8, N), jnp.float32),
        pltpu.VMEM((2, BLK), jnp.int32),
        pltpu.VMEM((2, BLK), jnp.int32),
        pltpu.VMEM((H, BLK), jnp.float32),
        pltpu.VMEM((H, BLK), jnp.float32),
        pltpu.VMEM((NSC, SROW), jnp.int32),
        pltpu.VMEM((BLK, 8), jnp.float32),
        pltpu.VMEM_SHARED((NPAD, 8), jnp.float32),
        pltpu.SemaphoreType.DMA,
        pltpu.SemaphoreType.DMA,
    ],
)(_sc_edge_kernel)


@jax.jit
def _run(nfeat, efeat, edge_index, W_node, b_node, W_edge, b_edge,
         attn_l, attn_r, attn_e):
    hp = jax.lax.Precision.HIGHEST
    al, ar, ae = attn_l[0], attn_r[0], attn_e[0]
    Wn3 = W_node.reshape(-1, H, F)
    Wl = jnp.einsum('khf,hf->kh', Wn3, al, precision=hp)
    Wr = jnp.einsum('khf,hf->kh', Wn3, ar, precision=hp)
    Wlr = jnp.concatenate([Wl, Wr], axis=1)
    bn3 = b_node.reshape(H, F)
    clr = jnp.concatenate([jnp.sum(bn3 * al, -1), jnp.sum(bn3 * ar, -1)])
    We = jnp.einsum('khf,hf->kh', W_edge.reshape(-1, H, F), ae,
                    precision=hp)
    ce = jnp.sum(b_edge.reshape(H, F) * ae, -1)

    nfeat_p = jnp.pad(nfeat, ((0, NPAD - N), (0, 0)))
    clr2 = jnp.tile(clr[:, None], (1, 128))
    ce2 = jnp.tile(ce[:, None], (1, 128))

    BN = 1024
    tab = pl.pallas_call(
        _node_proj_body,
        grid=(NPAD // BN,),
        in_specs=[
            pl.BlockSpec((128, 8), lambda i: (0, 0)),
            pl.BlockSpec((8, 128), lambda i: (0, 0)),
            pl.BlockSpec((BN, 128), lambda i: (i, 0)),
        ],
        out_specs=pl.BlockSpec((8, BN), lambda i: (0, i)),
        out_shape=jax.ShapeDtypeStruct((8, NPAD), jnp.float32),
    )(Wlr, clr2, nfeat_p)

    BE = 12800
    eeT = pl.pallas_call(
        _edge_proj_body,
        grid=(E // BE,),
        in_specs=[
            pl.BlockSpec((16, 4), lambda i: (0, 0)),
            pl.BlockSpec((4, 128), lambda i: (0, 0)),
            pl.BlockSpec((16, BE), lambda i: (0, i)),
        ],
        out_specs=pl.BlockSpec((4, BE), lambda i: (0, i)),
        out_shape=jax.ShapeDtypeStruct((4, E), jnp.float32),
    )(We, ce2, efeat.T)

    zero_rows = jnp.zeros((ACC_PT, 8), jnp.float32)

    part = _sc_edge_call(tab, edge_index, eeT, zero_rows)

    BF = 1024
    fin = pl.pallas_call(
        _fin_body,
        grid=(NPAD // BF,),
        in_specs=[
            pl.BlockSpec((BF, 8), lambda i: (i, 0)),
            pl.BlockSpec((BF, 8), lambda i: (i + NPAD // BF, 0)),
        ],
        out_specs=pl.BlockSpec((BF, 4), lambda i: (i, 0)),
        out_shape=jax.ShapeDtypeStruct((NPAD, 4), jnp.float32),
    )(part, part)

    return fin[:N].reshape(N, H, 1)


def kernel(nfeat, efeat, edge_index, W_node, b_node, W_edge, b_edge,
           attn_l, attn_r, attn_e):
    return _run(nfeat, efeat, edge_index, W_node, b_node, W_edge, b_edge,
                attn_l, attn_r, attn_e)


# speedup vs baseline: 1.2111x; 1.2111x over previous
import functools
import jax
import jax.numpy as jnp
from jax import lax
from jax.experimental import pallas as pl
from jax.experimental.pallas import tpu as pltpu
from jax.experimental.pallas import tpu_sc as plsc

H = 4
F = 32
NEG_SLOPE = 0.2

NC, NS, L = 2, 16, 16
NW = NC * NS

N = 10000
E = 320000
NPAD = 10240
EPW = E // NW
BLK = 2000
NBLK = EPW // BLK
GRP = BLK // L
SROW = 125
NSC = BLK // SROW
ACC_PT = NPAD // NS


def _node_proj_body(w_ref, c_ref, x_ref, o_ref):
    o_ref[...] = lax.dot_general(
        w_ref[...], x_ref[...], (((0,), (1,)), ((), ())),
        preferred_element_type=jnp.float32,
    ) + c_ref[...][:, :1]


def _edge_proj_body(w_ref, c_ref, x_ref, o_ref):
    o_ref[...] = lax.dot_general(
        w_ref[...], x_ref[...], (((0,), (0,)), ((), ())),
        preferred_element_type=jnp.float32,
    ) + c_ref[...][:, :1]


def _fin_body(a_ref, b_ref, o_ref):
    a = a_ref[...]
    b = b_ref[...]
    den = a[:, 0:4] + b[:, 0:4]
    ms = a[:, 4:8] + b[:, 4:8]
    o_ref[...] = jnp.where(den > 0, ms / den, 0.0)


def _sc_edge_kernel(tab_hbm, ei_hbm, ee_hbm, zero_hbm, out_hbm,
                    tab_v, sda_v, sdb_v, eea_v, eeb_v, dst2_v, buf_v, acc,
                    sem_a, sem_in):
    cid = lax.axis_index("c")
    sid = lax.axis_index("s")
    wid = cid * NS + sid

    pltpu.sync_copy(tab_hbm.at[:, pl.ds(0, N)], tab_v)

    pltpu.sync_copy(zero_hbm, acc.at[pl.ds(sid * ACC_PT, ACC_PT)])
    plsc.subcore_barrier()

    iota = lax.iota(jnp.int32, L)
    base = wid * EPW
    buf = buf_v
    dst2 = dst2_v
    sds = [sda_v, sdb_v]
    ees = [eea_v, eeb_v]

    def start_inputs(k):
        off = base + k * BLK
        return [
            pltpu.async_copy(ei_hbm.at[:, pl.ds(off, BLK)], sds[k % 2], sem_in),
            pltpu.async_copy(ee_hbm.at[:, pl.ds(off, BLK)], ees[k % 2], sem_in),
        ]

    descs = [None] * NBLK
    in_descs = [None] * NBLK
    in_descs[0] = start_inputs(0)
    for k in range(NBLK):
        for d in in_descs[k]:
            d.wait()
        if k + 1 < NBLK:
            in_descs[k + 1] = start_inputs(k + 1)
        if k >= 1:
            for d in descs[k - 1]:
                d.wait()
        sd_v = sds[k % 2]
        ee_v = ees[k % 2]

        @plsc.parallel_loop(0, GRP, unroll=4)
        def gbody(g):
            s16 = g * L
            src16 = sd_v[0, pl.ds(s16, L)]
            dst16 = sd_v[1, pl.ds(s16, L)]
            pos = s16 + iota
            plsc.store_scatter(dst2, [pos // SROW, pos % SROW], dst16)
            for h in range(H):
                hrow = jnp.full((L,), h, jnp.int32)
                el = plsc.load_gather(tab_v, [hrow, src16])
                er = plsc.load_gather(tab_v, [hrow + H, dst16])
                ee = ee_v[h, pl.ds(s16, L)]
                s = el + ee
                t = s + er
                t = jnp.where(t >= 0.0, t, NEG_SLOPE * t)
                num = jnp.exp(t)
                ns = num * s
                plsc.store_scatter(buf, [pos, hrow], num)
                plsc.store_scatter(buf, [pos, hrow + H], ns)

        descs[k] = [
            pltpu.async_copy(buf.at[pl.ds(j * SROW, SROW)],
                             acc.at[dst2.at[j]], sem_a, add=True)
            for j in range(NSC)
        ]

    for d in descs[NBLK - 1]:
        d.wait()

    plsc.subcore_barrier()
    r0 = sid * ACC_PT
    pltpu.sync_copy(acc.at[pl.ds(r0, ACC_PT)],
                    out_hbm.at[pl.ds(cid * NPAD + r0, ACC_PT)])


_sc_edge_call = functools.partial(
    pl.kernel,
    out_type=jax.ShapeDtypeStruct((NC * NPAD, 8), jnp.float32),
    mesh=plsc.VectorSubcoreMesh(core_axis_name="c", subcore_axis_name="s"),
    compiler_params=pltpu.CompilerParams(needs_layout_passes=False,
                                         use_tc_tiling_on_sc=False),
    scratch_types=[
        pltpu.VMEM((8, N), jnp.float32),
        pltpu.VMEM((2, BLK), jnp.int32),
        pltpu.VMEM((2, BLK), jnp.int32),
        pltpu.VMEM((H, BLK), jnp.float32),
        pltpu.VMEM((H, BLK), jnp.float32),
        pltpu.VMEM((NSC, SROW), jnp.int32),
        pltpu.VMEM((BLK, 8), jnp.float32),
        pltpu.VMEM_SHARED((NPAD, 8), jnp.float32),
        pltpu.SemaphoreType.DMA,
        pltpu.SemaphoreType.DMA,
    ],
)(_sc_edge_kernel)


@jax.jit
def _run(nfeat, efeat, edge_index, W_node, b_node, W_edge, b_edge,
         attn_l, attn_r, attn_e):
    hp = jax.lax.Precision.HIGHEST
    al, ar, ae = attn_l[0], attn_r[0], attn_e[0]
    Wn3 = W_node.reshape(-1, H, F)
    Wl = jnp.einsum('khf,hf->kh', Wn3, al, precision=hp)
    Wr = jnp.einsum('khf,hf->kh', Wn3, ar, precision=hp)
    Wlr = jnp.concatenate([Wl, Wr], axis=1)
    bn3 = b_node.reshape(H, F)
    clr = jnp.concatenate([jnp.sum(bn3 * al, -1), jnp.sum(bn3 * ar, -1)])
    We = jnp.einsum('khf,hf->kh', W_edge.reshape(-1, H, F), ae,
                    precision=hp)
    ce = jnp.sum(b_edge.reshape(H, F) * ae, -1)

    nfeat_p = jnp.pad(nfeat, ((0, NPAD - N), (0, 0)))
    clr2 = jnp.tile(clr[:, None], (1, 128))
    ce2 = jnp.tile(ce[:, None], (1, 128))

    BN = 1024
    tab = pl.pallas_call(
        _node_proj_body,
        grid=(NPAD // BN,),
        in_specs=[
            pl.BlockSpec((128, 8), lambda i: (0, 0)),
            pl.BlockSpec((8, 128), lambda i: (0, 0)),
            pl.BlockSpec((BN, 128), lambda i: (i, 0)),
        ],
        out_specs=pl.BlockSpec((8, BN), lambda i: (0, i)),
        out_shape=jax.ShapeDtypeStruct((8, NPAD), jnp.float32),
    )(Wlr, clr2, nfeat_p)

    BE = 12800
    eeT = pl.pallas_call(
        _edge_proj_body,
        grid=(E // BE,),
        in_specs=[
            pl.BlockSpec((16, 4), lambda i: (0, 0)),
            pl.BlockSpec((4, 128), lambda i: (0, 0)),
            pl.BlockSpec((16, BE), lambda i: (0, i)),
        ],
        out_specs=pl.BlockSpec((4, BE), lambda i: (0, i)),
        out_shape=jax.ShapeDtypeStruct((4, E), jnp.float32),
    )(We, ce2, efeat.T)

    zero_rows = jnp.zeros((ACC_PT, 8), jnp.float32)

    part = _sc_edge_call(tab, edge_index, eeT, zero_rows)

    BF = 1024
    fin = pl.pallas_call(
        _fin_body,
        grid=(NPAD // BF,),
        in_specs=[
            pl.BlockSpec((BF, 8), lambda i: (i, 0)),
            pl.BlockSpec((BF, 8), lambda i: (i + NPAD // BF, 0)),
        ],
        out_specs=pl.BlockSpec((BF, 4), lambda i: (i, 0)),
        out_shape=jax.ShapeDtypeStruct((NPAD, 4), jnp.float32),
    )(part, part)

    return fin[:N].reshape(N, H, 1)


def kernel(nfeat, efeat, edge_index, W_node, b_node, W_edge, b_edge,
           attn_l, attn_r, attn_e):
    return _run(nfeat, efeat, edge_index, W_node, b_node, W_edge, b_edge,
                attn_l, attn_r, attn_e)


# speedup vs baseline: 1.2293x; 1.0151x over previous
import functools
import jax
import jax.numpy as jnp
from jax import lax
from jax.experimental import pallas as pl
from jax.experimental.pallas import tpu as pltpu
from jax.experimental.pallas import tpu_sc as plsc

H = 4
F = 32
NEG_SLOPE = 0.2

NC, NS, L = 2, 16, 16
NW = NC * NS

N = 10000
E = 320000
NPAD = 10240
EPW = E // NW
BLK = 2000
NBLK = EPW // BLK
GRP = BLK // L
SROW = 125
NSC = BLK // SROW
ACC_PT = NPAD // NS


def _node_proj_body(w_ref, c_ref, x_ref, o_ref):
    o_ref[...] = lax.dot_general(
        w_ref[...], x_ref[...], (((0,), (1,)), ((), ())),
        preferred_element_type=jnp.float32,
    ) + c_ref[...][:, :1]


def _edge_proj_body(w_ref, c_ref, x_ref, o_ref):
    o_ref[...] = lax.dot_general(
        w_ref[...], x_ref[...], (((0,), (0,)), ((), ())),
        preferred_element_type=jnp.float32,
    ) + c_ref[...][:, :1]


def _fin_body(a_ref, b_ref, o_ref):
    a = a_ref[...]
    b = b_ref[...]
    den = a[:, 0:4] + b[:, 0:4]
    ms = a[:, 4:8] + b[:, 4:8]
    o_ref[...] = jnp.where(den > 0, ms / den, 0.0)


def _sc_edge_kernel(tab_hbm, ei_hbm, ee_hbm, zero_hbm, out_hbm,
                    tab_v, sda_v, sdb_v, eea_v, eeb_v, dst2_v, buf_v, acc,
                    sem_a, sem_in, sem_tz):
    cid = lax.axis_index("c")
    sid = lax.axis_index("s")
    wid = cid * NS + sid

    iota = lax.iota(jnp.int32, L)
    base = wid * EPW
    buf = buf_v
    dst2 = dst2_v
    sds = [sda_v, sdb_v]
    ees = [eea_v, eeb_v]

    def start_inputs(k):
        off = base + k * BLK
        return [
            pltpu.async_copy(ei_hbm.at[:, pl.ds(off, BLK)], sds[k % 2], sem_in),
            pltpu.async_copy(ee_hbm.at[:, pl.ds(off, BLK)], ees[k % 2], sem_in),
        ]

    descs = [None] * NBLK
    in_descs = [None] * NBLK
    in_descs[0] = start_inputs(0)
    tz = [
        pltpu.async_copy(tab_hbm.at[:, pl.ds(0, N)], tab_v, sem_tz),
        pltpu.async_copy(zero_hbm, acc.at[pl.ds(sid * ACC_PT, ACC_PT)], sem_tz),
    ]
    for d in tz:
        d.wait()
    plsc.subcore_barrier()
    for k in range(NBLK):
        for d in in_descs[k]:
            d.wait()
        if k + 1 < NBLK:
            in_descs[k + 1] = start_inputs(k + 1)
        if k >= 1:
            for d in descs[k - 1]:
                d.wait()
        sd_v = sds[k % 2]
        ee_v = ees[k % 2]

        @plsc.parallel_loop(0, GRP, unroll=4)
        def gbody(g):
            s16 = g * L
            src16 = sd_v[0, pl.ds(s16, L)]
            dst16 = sd_v[1, pl.ds(s16, L)]
            pos = s16 + iota
            plsc.store_scatter(dst2, [pos // SROW, pos % SROW], dst16)
            for h in range(H):
                hrow = jnp.full((L,), h, jnp.int32)
                el = plsc.load_gather(tab_v, [hrow, src16])
                er = plsc.load_gather(tab_v, [hrow + H, dst16])
                ee = ee_v[h, pl.ds(s16, L)]
                s = el + ee
                t = s + er
                t = jnp.where(t >= 0.0, t, NEG_SLOPE * t)
                num = jnp.exp(t)
                ns = num * s
                plsc.store_scatter(buf, [pos, hrow], num)
                plsc.store_scatter(buf, [pos, hrow + H], ns)

        descs[k] = [
            pltpu.async_copy(buf.at[pl.ds(j * SROW, SROW)],
                             acc.at[dst2.at[j]], sem_a, add=True)
            for j in range(NSC)
        ]

    for d in descs[NBLK - 1]:
        d.wait()

    plsc.subcore_barrier()
    r0 = sid * ACC_PT
    pltpu.sync_copy(acc.at[pl.ds(r0, ACC_PT)],
                    out_hbm.at[pl.ds(cid * NPAD + r0, ACC_PT)])


_sc_edge_call = functools.partial(
    pl.kernel,
    out_type=jax.ShapeDtypeStruct((NC * NPAD, 8), jnp.float32),
    mesh=plsc.VectorSubcoreMesh(core_axis_name="c", subcore_axis_name="s"),
    compiler_params=pltpu.CompilerParams(needs_layout_passes=False,
                                         use_tc_tiling_on_sc=False),
    scratch_types=[
        pltpu.VMEM((8, N), jnp.float32),
        pltpu.VMEM((2, BLK), jnp.int32),
        pltpu.VMEM((2, BLK), jnp.int32),
        pltpu.VMEM((H, BLK), jnp.float32),
        pltpu.VMEM((H, BLK), jnp.float32),
        pltpu.VMEM((NSC, SROW), jnp.int32),
        pltpu.VMEM((BLK, 8), jnp.float32),
        pltpu.VMEM_SHARED((NPAD, 8), jnp.float32),
        pltpu.SemaphoreType.DMA,
        pltpu.SemaphoreType.DMA,
        pltpu.SemaphoreType.DMA,
    ],
)(_sc_edge_kernel)


@jax.jit
def _run(nfeat, efeat, edge_index, W_node, b_node, W_edge, b_edge,
         attn_l, attn_r, attn_e):
    hp = jax.lax.Precision.HIGHEST
    al, ar, ae = attn_l[0], attn_r[0], attn_e[0]
    Wn3 = W_node.reshape(-1, H, F)
    Wl = jnp.einsum('khf,hf->kh', Wn3, al, precision=hp)
    Wr = jnp.einsum('khf,hf->kh', Wn3, ar, precision=hp)
    Wlr = jnp.concatenate([Wl, Wr], axis=1)
    bn3 = b_node.reshape(H, F)
    clr = jnp.concatenate([jnp.sum(bn3 * al, -1), jnp.sum(bn3 * ar, -1)])
    We = jnp.einsum('khf,hf->kh', W_edge.reshape(-1, H, F), ae,
                    precision=hp)
    ce = jnp.sum(b_edge.reshape(H, F) * ae, -1)

    nfeat_p = jnp.pad(nfeat, ((0, NPAD - N), (0, 0)))
    clr2 = jnp.tile(clr[:, None], (1, 128))
    ce2 = jnp.tile(ce[:, None], (1, 128))

    BN = 1024
    tab = pl.pallas_call(
        _node_proj_body,
        grid=(NPAD // BN,),
        in_specs=[
            pl.BlockSpec((128, 8), lambda i: (0, 0)),
            pl.BlockSpec((8, 128), lambda i: (0, 0)),
            pl.BlockSpec((BN, 128), lambda i: (i, 0)),
        ],
        out_specs=pl.BlockSpec((8, BN), lambda i: (0, i)),
        out_shape=jax.ShapeDtypeStruct((8, NPAD), jnp.float32),
    )(Wlr, clr2, nfeat_p)

    BE = 12800
    eeT = pl.pallas_call(
        _edge_proj_body,
        grid=(E // BE,),
        in_specs=[
            pl.BlockSpec((16, 4), lambda i: (0, 0)),
            pl.BlockSpec((4, 128), lambda i: (0, 0)),
            pl.BlockSpec((16, BE), lambda i: (0, i)),
        ],
        out_specs=pl.BlockSpec((4, BE), lambda i: (0, i)),
        out_shape=jax.ShapeDtypeStruct((4, E), jnp.float32),
    )(We, ce2, efeat.T)

    zero_rows = jnp.zeros((ACC_PT, 8), jnp.float32)

    part = _sc_edge_call(tab, edge_index, eeT, zero_rows)

    BF = 1024
    fin = pl.pallas_call(
        _fin_body,
        grid=(NPAD // BF,),
        in_specs=[
            pl.BlockSpec((BF, 8), lambda i: (i, 0)),
            pl.BlockSpec((BF, 8), lambda i: (i + NPAD // BF, 0)),
        ],
        out_specs=pl.BlockSpec((BF, 4), lambda i: (i, 0)),
        out_shape=jax.ShapeDtypeStruct((NPAD, 4), jnp.float32),
    )(part, part)

    return fin[:N].reshape(N, H, 1)


def kernel(nfeat, efeat, edge_index, W_node, b_node, W_edge, b_edge,
           attn_l, attn_r, attn_e):
    return _run(nfeat, efeat, edge_index, W_node, b_node, W_edge, b_edge,
                attn_l, attn_r, attn_e)
